# baseline (speedup 1.0000x reference)
import functools

import jax
import jax.numpy as jnp
from jax import lax
from jax.experimental import pallas as pl
from jax.experimental.pallas import tpu as pltpu
from jax.experimental.pallas import tpu_sc as plsc

CH = 8
NW = 32
L = 16
OG = 8


def _scalar_at_i32(ref, j):
    b = plsc.load_gather(ref, [jnp.full((L,), j, jnp.int32)])
    return jnp.squeeze(lax.slice(b, (0,), (1,)))


def _scalar_at_f32(ref, j):
    b = plsc.load_gather(ref, [jnp.full((L,), j, jnp.int32)])
    return jnp.squeeze(lax.slice(b, (0,), (1,)))


def _sc_body(D, WPW,
             feats_hbm, gb_hbm, n_hbm, coef_hbm, pos_hbm, ptab_hbm,
             out_hbm,
             gb_v, n_v, coef_v, pos_v, ptab_v, buf_v, outbuf_v):
    nsl = D // L
    wid = lax.axis_index("s") * 2 + lax.axis_index("c")
    base = wid * WPW
    pltpu.sync_copy(gb_hbm.at[pl.ds(base, WPW)], gb_v)
    pltpu.sync_copy(n_hbm.at[pl.ds(base, WPW)], n_v)
    pltpu.sync_copy(coef_hbm.at[pl.ds(base, WPW)], coef_v)
    pltpu.sync_copy(pos_hbm.at[pl.ds(base, WPW)], pos_v)
    pltpu.sync_copy(ptab_hbm, ptab_v)

    zero16 = jnp.zeros((L,), jnp.float32)

    def word(j2, jbase):
        j = jbase + j2
        gb = _scalar_at_i32(gb_v, j)
        n = _scalar_at_i32(n_v, j)
        p = _scalar_at_i32(pos_v, j)
        cf = _scalar_at_f32(coef_v, j)

        hi = gb + n
        a0 = gb - lax.rem(gb, CH)
        nch = (hi - a0 + (CH - 1)) // CH

        def chunk(c, acc):
            a = pl.multiple_of(a0 + c * CH, CH)
            pltpu.sync_copy(feats_hbm.at[pl.ds(a, CH)], buf_v)
            for r in range(CH):
                g = a + r
                w = jnp.where((g >= gb) & (g < hi), jnp.float32(1), jnp.float32(0))
                wv = jnp.full((L,), w, jnp.float32)
                acc = tuple(acc[v] + buf_v[r, pl.ds(v * L, L)] * wv
                            for v in range(nsl))
            return acc

        acc = lax.fori_loop(0, nch, chunk, (zero16,) * nsl)

        cfv = jnp.full((L,), cf, jnp.float32)
        pbase = p * D
        obase = j2 * D
        for v in range(nsl):
            prow = ptab_v[pl.ds(pbase + v * L, L)]
            outbuf_v[pl.ds(obase + v * L, L)] = acc[v] * cfv + prow
        return jbase

    def group(g, _):
        jbase = g * OG
        lax.fori_loop(0, OG, word, jbase)
        ob = pl.multiple_of((base + jbase) * D, 8)
        pltpu.sync_copy(outbuf_v, out_hbm.at[pl.ds(ob, OG * D)])
        return _

    lax.fori_loop(0, WPW // OG, group, 0)


def kernel(feats, word_lens, seq_len, pos, pos_table):
    B, S, D = feats.shape
    W = word_lens.shape[1]
    PV = pos_table.shape[0]
    WPW = (B * W) // NW

    wl = word_lens.astype(jnp.int32)
    nxt = jnp.concatenate([wl[:, 1:], jnp.zeros((B, 1), jnp.int32)], axis=1)
    end = jnp.where(nxt == 0, seq_len[:, None].astype(jnp.int32), nxt)
    start = jnp.clip(wl, 0, S)
    end = jnp.clip(end, 0, S)
    lo = jnp.minimum(start, end)
    n = jnp.maximum(start, end) - lo
    jidx = jnp.arange(W, dtype=jnp.int32)[None, :]
    valid = ~((wl == 0) & (jidx != 0))
    coef = jnp.where(end > start,
                     1.0 / jnp.maximum(end - start, 1).astype(jnp.float32),
                     jnp.float32(-1))
    coef = jnp.where(valid, coef, 0.0).astype(jnp.float32)

    ibase = (jnp.arange(B, dtype=jnp.int32) * S)[:, None]
    gb = (lo + ibase).reshape(-1)
    nf = n.reshape(-1)
    cf = coef.reshape(-1)
    pf = pos.reshape(-1).astype(jnp.int32)
    feats2 = feats.reshape(B * S, D)
    ptab_flat = pos_table.reshape(PV * D)

    mesh = plsc.VectorSubcoreMesh(core_axis_name="c", subcore_axis_name="s",
                                  num_cores=2, num_subcores=16)
    fn = functools.partial(
        pl.kernel,
        out_type=jax.ShapeDtypeStruct((B * W * D,), jnp.float32),
        mesh=mesh,
        compiler_params=pltpu.CompilerParams(needs_layout_passes=False),
        scratch_types=[
            pltpu.VMEM((WPW,), jnp.int32),
            pltpu.VMEM((WPW,), jnp.int32),
            pltpu.VMEM((WPW,), jnp.float32),
            pltpu.VMEM((WPW,), jnp.int32),
            pltpu.VMEM((PV * D,), jnp.float32),
            pltpu.VMEM((CH, D), jnp.float32),
            pltpu.VMEM((OG * D,), jnp.float32),
        ],
    )(functools.partial(_sc_body, D, WPW))
    out = fn(feats2, gb, nf, cf, pf, ptab_flat)
    return out.reshape(B, W, D)
